# Initial kernel scaffold; baseline (speedup 1.0000x reference)
import jax
import jax.numpy as jnp
from jax.experimental import pallas as pl


def kernel(node_vectors, batch_idx, global_vectors, W_node, W_global):
    raise NotImplementedError("write your pallas kernel here")


# speedup vs baseline: 6.7772x; 6.7772x over previous
import jax
import jax.numpy as jnp
from jax import lax
from jax.experimental import pallas as pl
from jax.experimental.pallas import tpu as pltpu

N = 50000
B = 256
H = 256
HF = 3 * H

KN = 1000
NBLK = N // KN
NPAD = 50176

IDXR = NPAD // 128
IDXT = IDXR // 8


def _onehot_bf16(idxb):
    oh = (idxb[:, None] == lax.broadcasted_iota(jnp.int32, (1, B), 1))
    return oh.astype(jnp.bfloat16)


def _pass1_body(x_ref, idx_ref, sums_ref, acc_ref):
    i = pl.program_id(0)

    @pl.when(i == 0)
    def _():
        acc_ref[...] = jnp.zeros_like(acc_ref)

    p = _onehot_bf16(idx_ref[0, 0, :])
    xc = x_ref[...].astype(jnp.bfloat16)
    acc_ref[...] += lax.dot_general(p, xc, (((0,), (0,)), ((), ())),
                                    preferred_element_type=jnp.float32)

    @pl.when(i == NBLK - 1)
    def _():
        sums_ref[...] = acc_ref[...]


_pass1 = pl.pallas_call(
    _pass1_body,
    grid=(NBLK,),
    in_specs=[
        pl.BlockSpec((KN, HF), lambda i: (i, 0)),
        pl.BlockSpec((1, 1, KN), lambda i: (i, 0, 0)),
    ],
    out_specs=pl.BlockSpec((B, HF), lambda i: (0, 0)),
    out_shape=jax.ShapeDtypeStruct((B, HF), jnp.float32),
    scratch_shapes=[pltpu.VMEM((B, HF), jnp.float32)],
)


def _prep_body(sums_ref, idx2d_ref, gvf_ref, wg_ref, wn_ref, ng_ref, gw_ref):
    sums = sums_ref[...]

    iota_b = lax.broadcasted_iota(jnp.int32, (1, 1, B), 2)

    def cnt_body(t, cnt):
        blk = idx2d_ref[pl.ds(t * 8, 8), :]
        eq = (blk[:, :, None] == iota_b).astype(jnp.float32)
        return cnt + jnp.sum(eq, axis=(0, 1))[None, :]

    cnt_row = lax.fori_loop(0, IDXT, cnt_body,
                            jnp.zeros((1, B), jnp.float32))
    eye = (lax.broadcasted_iota(jnp.int32, (B, B), 0)
           == lax.broadcasted_iota(jnp.int32, (B, B), 1)).astype(jnp.float32)
    cnt_col = lax.dot_general(eye, cnt_row, (((1,), (1,)), ((), ())),
                              preferred_element_type=jnp.float32)
    inv = 1.0 / jnp.maximum(cnt_col, 1.0)

    t = sums * inv + gvf_ref[...]
    for j in range(3):
        tj = t[:, j * H:(j + 1) * H]
        ngj = lax.dot_general(tj, wg_ref[...], (((1,), (1,)), ((), ())),
                              preferred_element_type=jnp.float32)
        ng_ref[:, j * H:(j + 1) * H] = ngj
        gw_ref[:, j * H:(j + 1) * H] = lax.dot_general(
            ngj, wn_ref[...], (((1,), (1,)), ((), ())),
            preferred_element_type=jnp.float32)


_prep = pl.pallas_call(
    _prep_body,
    out_shape=(
        jax.ShapeDtypeStruct((B, HF), jnp.float32),
        jax.ShapeDtypeStruct((B, HF), jnp.float32),
    ),
)


def _main_body(x_ref, idx_ref, gw_ref, wn_ref, o_ref):
    xb = x_ref[...]
    xc = xb.astype(jnp.bfloat16)
    wn = wn_ref[...].astype(jnp.bfloat16)
    mms = []
    for j in range(3):
        xj = xc[:, j * H:(j + 1) * H]
        mms.append(lax.dot_general(xj, wn, (((1,), (1,)), ((), ())),
                                   preferred_element_type=jnp.float32))
    mm = jnp.concatenate(mms, axis=1)
    p = _onehot_bf16(idx_ref[0, 0, :])
    gw2 = gw_ref[...].astype(jnp.bfloat16)
    g = lax.dot_general(p, gw2, (((1,), (0,)), ((), ())),
                        preferred_element_type=jnp.float32)
    o_ref[...] = mm + xb + g


_main = pl.pallas_call(
    _main_body,
    grid=(NBLK,),
    in_specs=[
        pl.BlockSpec((KN, HF), lambda i: (i, 0)),
        pl.BlockSpec((1, 1, KN), lambda i: (i, 0, 0)),
        pl.BlockSpec((B, HF), lambda i: (0, 0)),
        pl.BlockSpec((H, H), lambda i: (0, 0)),
    ],
    out_specs=pl.BlockSpec((KN, HF), lambda i: (i, 0)),
    out_shape=jax.ShapeDtypeStruct((N, HF), jnp.float32),
)


def kernel(node_vectors, batch_idx, global_vectors, W_node, W_global):
    idx32 = batch_idx.astype(jnp.int32)
    idx_pad = jnp.pad(idx32, (0, NPAD - N), constant_values=B)
    idx3 = idx32.reshape(NBLK, 1, KN)
    x2d = node_vectors.reshape(N, HF)

    sums = _pass1(x2d, idx3)

    gv_flat = global_vectors.reshape(B, HF)
    new_global, gw = _prep(sums, idx_pad.reshape(IDXR, 128), gv_flat,
                           W_global, W_node)

    enhanced = _main(x2d, idx3, gw, W_node)
    return enhanced.reshape(N, 3, H), new_global.reshape(B, 3, H)


# speedup vs baseline: 24.6775x; 3.6412x over previous
import jax
import jax.numpy as jnp
from jax import lax
from jax.experimental import pallas as pl
from jax.experimental.pallas import tpu as pltpu

N = 50000
B = 256
H = 256

KN = 1000
NBLK = N // KN


def _onehot_bf16(idxb):
    oh = (idxb[:, None] == lax.broadcasted_iota(jnp.int32, (1, B), 1))
    return oh.astype(jnp.bfloat16)


def _pass1_body(x_ref, idx_ref, gv_ref, wg_ref, wn_ref,
                ng_ref, gw_ref, acc_ref, cnt_ref):
    i = pl.program_id(0)

    @pl.when(i == 0)
    def _():
        acc_ref[...] = jnp.zeros_like(acc_ref)
        cnt_ref[...] = jnp.zeros_like(cnt_ref)

    p = _onehot_bf16(idx_ref[0, 0, :])
    for j in range(3):
        xc = x_ref[j].astype(jnp.bfloat16)
        acc_ref[j] += lax.dot_general(p, xc, (((0,), (0,)), ((), ())),
                                      preferred_element_type=jnp.float32)
    ones = jnp.ones((KN, 1), jnp.bfloat16)
    cnt_ref[...] += lax.dot_general(p, ones, (((0,), (0,)), ((), ())),
                                    preferred_element_type=jnp.float32)

    @pl.when(i == NBLK - 1)
    def _():
        inv = 1.0 / jnp.maximum(cnt_ref[...], 1.0)
        wg = wg_ref[...]
        wn = wn_ref[...]
        for j in range(3):
            tj = acc_ref[j] * inv + gv_ref[j]
            ngj = lax.dot_general(tj, wg, (((1,), (1,)), ((), ())),
                                  preferred_element_type=jnp.float32)
            ng_ref[j] = ngj
            gw_ref[j] = lax.dot_general(ngj, wn, (((1,), (1,)), ((), ())),
                                        preferred_element_type=jnp.float32)


_pass1 = pl.pallas_call(
    _pass1_body,
    grid=(NBLK,),
    in_specs=[
        pl.BlockSpec((3, KN, H), lambda i: (0, i, 0)),
        pl.BlockSpec((1, 1, KN), lambda i: (i, 0, 0)),
        pl.BlockSpec((3, B, H), lambda i: (0, 0, 0)),
        pl.BlockSpec((H, H), lambda i: (0, 0)),
        pl.BlockSpec((H, H), lambda i: (0, 0)),
    ],
    out_specs=(
        pl.BlockSpec((3, B, H), lambda i: (0, 0, 0)),
        pl.BlockSpec((3, B, H), lambda i: (0, 0, 0)),
    ),
    out_shape=(
        jax.ShapeDtypeStruct((3, B, H), jnp.float32),
        jax.ShapeDtypeStruct((3, B, H), jnp.float32),
    ),
    scratch_shapes=[pltpu.VMEM((3, B, H), jnp.float32),
                    pltpu.VMEM((B, 1), jnp.float32)],
)


def _main_body(x_ref, idx_ref, gw_ref, wn_ref, o_ref):
    p = _onehot_bf16(idx_ref[0, 0, :])
    wn = wn_ref[...].astype(jnp.bfloat16)
    for j in range(3):
        xj = x_ref[j]
        mm = lax.dot_general(xj.astype(jnp.bfloat16), wn,
                             (((1,), (1,)), ((), ())),
                             preferred_element_type=jnp.float32)
        g = lax.dot_general(p, gw_ref[j].astype(jnp.bfloat16),
                            (((1,), (0,)), ((), ())),
                            preferred_element_type=jnp.float32)
        o_ref[j] = mm + xj + g


_main = pl.pallas_call(
    _main_body,
    grid=(NBLK,),
    in_specs=[
        pl.BlockSpec((3, KN, H), lambda i: (0, i, 0)),
        pl.BlockSpec((1, 1, KN), lambda i: (i, 0, 0)),
        pl.BlockSpec((3, B, H), lambda i: (0, 0, 0)),
        pl.BlockSpec((H, H), lambda i: (0, 0)),
    ],
    out_specs=pl.BlockSpec((3, KN, H), lambda i: (0, i, 0)),
    out_shape=jax.ShapeDtypeStruct((3, N, H), jnp.float32),
)


def kernel(node_vectors, batch_idx, global_vectors, W_node, W_global):
    idx32 = batch_idx.astype(jnp.int32)
    idx3 = idx32.reshape(NBLK, 1, KN)
    x_t = node_vectors.transpose(1, 0, 2)
    gv_t = global_vectors.transpose(1, 0, 2)

    ng_t, gw_t = _pass1(x_t, idx3, gv_t, W_global, W_node)
    enh_t = _main(x_t, idx3, gw_t, W_node)
    return enh_t.transpose(1, 0, 2), ng_t.transpose(1, 0, 2)


# speedup vs baseline: 28.3626x; 1.1493x over previous
import jax
import jax.numpy as jnp
from jax import lax
from jax.experimental import pallas as pl
from jax.experimental.pallas import tpu as pltpu

N = 50000
B = 256
H = 256

KN = 2000
NBLK = N // KN


def _onehot_bf16(idxb):
    oh = (idxb[:, None] == lax.broadcasted_iota(jnp.int32, (1, B), 1))
    return oh.astype(jnp.bfloat16)


def _pass1_body(x_ref, idx_ref, gv_ref, wg_ref, wn_ref,
                ng_ref, gw_ref, acc_ref, cnt_ref):
    i = pl.program_id(0)

    @pl.when(i == 0)
    def _():
        acc_ref[...] = jnp.zeros_like(acc_ref)
        cnt_ref[...] = jnp.zeros_like(cnt_ref)

    p = _onehot_bf16(idx_ref[0, 0, :])
    for j in range(3):
        xc = x_ref[j].astype(jnp.bfloat16)
        acc_ref[j] += lax.dot_general(p, xc, (((0,), (0,)), ((), ())),
                                      preferred_element_type=jnp.float32)
    ones = jnp.ones((KN, 1), jnp.bfloat16)
    cnt_ref[...] += lax.dot_general(p, ones, (((0,), (0,)), ((), ())),
                                    preferred_element_type=jnp.float32)

    @pl.when(i == NBLK - 1)
    def _():
        inv = 1.0 / jnp.maximum(cnt_ref[...], 1.0)
        wg = wg_ref[...]
        wn = wn_ref[...]
        for j in range(3):
            tj = acc_ref[j] * inv + gv_ref[j]
            ngj = lax.dot_general(tj, wg, (((1,), (1,)), ((), ())),
                                  preferred_element_type=jnp.float32)
            ng_ref[j] = ngj
            gw_ref[j] = lax.dot_general(ngj, wn, (((1,), (1,)), ((), ())),
                                        preferred_element_type=jnp.float32)


_pass1 = pl.pallas_call(
    _pass1_body,
    grid=(NBLK,),
    in_specs=[
        pl.BlockSpec((3, KN, H), lambda i: (0, i, 0)),
        pl.BlockSpec((1, 1, KN), lambda i: (i, 0, 0)),
        pl.BlockSpec((3, B, H), lambda i: (0, 0, 0)),
        pl.BlockSpec((H, H), lambda i: (0, 0)),
        pl.BlockSpec((H, H), lambda i: (0, 0)),
    ],
    out_specs=(
        pl.BlockSpec((3, B, H), lambda i: (0, 0, 0)),
        pl.BlockSpec((3, B, H), lambda i: (0, 0, 0)),
    ),
    out_shape=(
        jax.ShapeDtypeStruct((3, B, H), jnp.float32),
        jax.ShapeDtypeStruct((3, B, H), jnp.float32),
    ),
    scratch_shapes=[pltpu.VMEM((3, B, H), jnp.float32),
                    pltpu.VMEM((B, 1), jnp.float32)],
)


def _main_body(x_ref, idx_ref, gw_ref, wn_ref, o_ref):
    p = _onehot_bf16(idx_ref[0, 0, :])
    wn = wn_ref[...].astype(jnp.bfloat16)
    for j in range(3):
        xj = x_ref[j]
        mm = lax.dot_general(xj.astype(jnp.bfloat16), wn,
                             (((1,), (1,)), ((), ())),
                             preferred_element_type=jnp.float32)
        g = lax.dot_general(p, gw_ref[j].astype(jnp.bfloat16),
                            (((1,), (0,)), ((), ())),
                            preferred_element_type=jnp.float32)
        o_ref[j] = mm + xj + g


_main = pl.pallas_call(
    _main_body,
    grid=(NBLK,),
    in_specs=[
        pl.BlockSpec((3, KN, H), lambda i: (0, i, 0)),
        pl.BlockSpec((1, 1, KN), lambda i: (i, 0, 0)),
        pl.BlockSpec((3, B, H), lambda i: (0, 0, 0)),
        pl.BlockSpec((H, H), lambda i: (0, 0)),
    ],
    out_specs=pl.BlockSpec((3, KN, H), lambda i: (0, i, 0)),
    out_shape=jax.ShapeDtypeStruct((3, N, H), jnp.float32),
)


def kernel(node_vectors, batch_idx, global_vectors, W_node, W_global):
    idx32 = batch_idx.astype(jnp.int32)
    idx3 = idx32.reshape(NBLK, 1, KN)
    x_t = node_vectors.transpose(1, 0, 2)
    gv_t = global_vectors.transpose(1, 0, 2)

    ng_t, gw_t = _pass1(x_t, idx3, gv_t, W_global, W_node)
    enh_t = _main(x_t, idx3, gw_t, W_node)
    return enh_t.transpose(1, 0, 2), ng_t.transpose(1, 0, 2)


# speedup vs baseline: 29.4756x; 1.0392x over previous
import jax
import jax.numpy as jnp
from jax import lax
from jax.experimental import pallas as pl
from jax.experimental.pallas import tpu as pltpu

N = 50000
B = 256
H = 256

KN = 2000
NBLK = N // KN
KP = 5000
NBLKP = N // KP


def _onehot_bf16(idxb):
    oh = (idxb[:, None] == lax.broadcasted_iota(jnp.int32, (1, B), 1))
    return oh.astype(jnp.bfloat16)


def _pass1_body(x_ref, idx_ref, gv_ref, wg_ref, wn_ref,
                ng_ref, gw_ref, acc_ref, cnt_ref):
    i = pl.program_id(0)

    @pl.when(i == 0)
    def _():
        acc_ref[...] = jnp.zeros_like(acc_ref)
        cnt_ref[...] = jnp.zeros_like(cnt_ref)

    p = _onehot_bf16(idx_ref[0, 0, :])
    for j in range(3):
        xc = x_ref[j].astype(jnp.bfloat16)
        acc_ref[j] += lax.dot_general(p, xc, (((0,), (0,)), ((), ())),
                                      preferred_element_type=jnp.float32)
    ones = jnp.ones((KP, 1), jnp.bfloat16)
    cnt_ref[...] += lax.dot_general(p, ones, (((0,), (0,)), ((), ())),
                                    preferred_element_type=jnp.float32)

    @pl.when(i == NBLKP - 1)
    def _():
        inv = 1.0 / jnp.maximum(cnt_ref[...], 1.0)
        wg = wg_ref[...]
        wn = wn_ref[...]
        for j in range(3):
            tj = acc_ref[j] * inv + gv_ref[j]
            ngj = lax.dot_general(tj, wg, (((1,), (1,)), ((), ())),
                                  preferred_element_type=jnp.float32)
            ng_ref[j] = ngj
            gw_ref[j] = lax.dot_general(ngj, wn, (((1,), (1,)), ((), ())),
                                        preferred_element_type=jnp.float32)


_pass1 = pl.pallas_call(
    _pass1_body,
    grid=(NBLKP,),
    in_specs=[
        pl.BlockSpec((3, KP, H), lambda i: (0, i, 0)),
        pl.BlockSpec((1, 1, KP), lambda i: (i, 0, 0)),
        pl.BlockSpec((3, B, H), lambda i: (0, 0, 0)),
        pl.BlockSpec((H, H), lambda i: (0, 0)),
        pl.BlockSpec((H, H), lambda i: (0, 0)),
    ],
    out_specs=(
        pl.BlockSpec((3, B, H), lambda i: (0, 0, 0)),
        pl.BlockSpec((3, B, H), lambda i: (0, 0, 0)),
    ),
    out_shape=(
        jax.ShapeDtypeStruct((3, B, H), jnp.float32),
        jax.ShapeDtypeStruct((3, B, H), jnp.float32),
    ),
    scratch_shapes=[pltpu.VMEM((3, B, H), jnp.float32),
                    pltpu.VMEM((B, 1), jnp.float32)],
)


def _main_body(x_ref, idx_ref, gw_ref, wn_ref, o_ref):
    p = _onehot_bf16(idx_ref[0, 0, :])
    wn = wn_ref[...].astype(jnp.bfloat16)
    for j in range(3):
        xj = x_ref[j]
        mm = lax.dot_general(xj.astype(jnp.bfloat16), wn,
                             (((1,), (1,)), ((), ())),
                             preferred_element_type=jnp.float32)
        g = lax.dot_general(p, gw_ref[j].astype(jnp.bfloat16),
                            (((1,), (0,)), ((), ())),
                            preferred_element_type=jnp.float32)
        o_ref[j] = mm + xj + g


_main = pl.pallas_call(
    _main_body,
    grid=(NBLK,),
    in_specs=[
        pl.BlockSpec((3, KN, H), lambda i: (0, i, 0)),
        pl.BlockSpec((1, 1, KN), lambda i: (i, 0, 0)),
        pl.BlockSpec((3, B, H), lambda i: (0, 0, 0)),
        pl.BlockSpec((H, H), lambda i: (0, 0)),
    ],
    out_specs=pl.BlockSpec((3, KN, H), lambda i: (0, i, 0)),
    out_shape=jax.ShapeDtypeStruct((3, N, H), jnp.float32),
)


def kernel(node_vectors, batch_idx, global_vectors, W_node, W_global):
    idx32 = batch_idx.astype(jnp.int32)
    idx3 = idx32.reshape(NBLK, 1, KN)
    idx3p = idx32.reshape(NBLKP, 1, KP)
    x_t = node_vectors.transpose(1, 0, 2)
    gv_t = global_vectors.transpose(1, 0, 2)

    ng_t, gw_t = _pass1(x_t, idx3p, gv_t, W_global, W_node)
    enh_t = _main(x_t, idx3, gw_t, W_node)
    return enh_t.transpose(1, 0, 2), ng_t.transpose(1, 0, 2)


# speedup vs baseline: 29.5034x; 1.0009x over previous
import jax
import jax.numpy as jnp
from jax import lax
from jax.experimental import pallas as pl
from jax.experimental.pallas import tpu as pltpu

N = 50000
B = 256
H = 256

KN = 2000
NBLK = N // KN


def _onehot_bf16(idxb):
    oh = (idxb[:, None] == lax.broadcasted_iota(jnp.int32, (1, B), 1))
    return oh.astype(jnp.bfloat16)


def _fused_body(x_ref, idx_ref, gv_ref, wg_ref, wn_ref,
                ng_ref, o_ref, acc_ref, cnt_ref, gw_ref):
    i = pl.program_id(0)

    @pl.when(i == 0)
    def _():
        acc_ref[...] = jnp.zeros_like(acc_ref)
        cnt_ref[...] = jnp.zeros_like(cnt_ref)

    p = _onehot_bf16(idx_ref[0, 0, :])

    @pl.when(i < NBLK)
    def _():
        for j in range(3):
            xc = x_ref[j].astype(jnp.bfloat16)
            acc_ref[j] += lax.dot_general(p, xc, (((0,), (0,)), ((), ())),
                                          preferred_element_type=jnp.float32)
        ones = jnp.ones((KN, 1), jnp.bfloat16)
        cnt_ref[...] += lax.dot_general(p, ones, (((0,), (0,)), ((), ())),
                                        preferred_element_type=jnp.float32)

    @pl.when(i == NBLK - 1)
    def _():
        inv = 1.0 / jnp.maximum(cnt_ref[...], 1.0)
        wg = wg_ref[...]
        wn = wn_ref[...]
        for j in range(3):
            tj = acc_ref[j] * inv + gv_ref[j]
            ngj = lax.dot_general(tj, wg, (((1,), (1,)), ((), ())),
                                  preferred_element_type=jnp.float32)
            ng_ref[j] = ngj
            gw_ref[j] = lax.dot_general(ngj, wn, (((1,), (1,)), ((), ())),
                                        preferred_element_type=jnp.float32)

    @pl.when(i >= NBLK)
    def _():
        wn = wn_ref[...].astype(jnp.bfloat16)
        for j in range(3):
            xj = x_ref[j]
            mm = lax.dot_general(xj.astype(jnp.bfloat16), wn,
                                 (((1,), (1,)), ((), ())),
                                 preferred_element_type=jnp.float32)
            g = lax.dot_general(p, gw_ref[j].astype(jnp.bfloat16),
                                (((1,), (0,)), ((), ())),
                                preferred_element_type=jnp.float32)
            o_ref[j] = mm + xj + g


_fused = pl.pallas_call(
    _fused_body,
    grid=(2 * NBLK,),
    in_specs=[
        pl.BlockSpec((3, KN, H), lambda i: (0, lax.rem(i, NBLK), 0)),
        pl.BlockSpec((1, 1, KN), lambda i: (lax.rem(i, NBLK), 0, 0)),
        pl.BlockSpec((3, B, H), lambda i: (0, 0, 0)),
        pl.BlockSpec((H, H), lambda i: (0, 0)),
        pl.BlockSpec((H, H), lambda i: (0, 0)),
    ],
    out_specs=(
        pl.BlockSpec((3, B, H), lambda i: (0, 0, 0)),
        pl.BlockSpec((3, KN, H),
                     lambda i: (0, jnp.maximum(i - NBLK, 0), 0)),
    ),
    out_shape=(
        jax.ShapeDtypeStruct((3, B, H), jnp.float32),
        jax.ShapeDtypeStruct((3, N, H), jnp.float32),
    ),
    scratch_shapes=[pltpu.VMEM((3, B, H), jnp.float32),
                    pltpu.VMEM((B, 1), jnp.float32),
                    pltpu.VMEM((3, B, H), jnp.float32)],
)


def kernel(node_vectors, batch_idx, global_vectors, W_node, W_global):
    idx32 = batch_idx.astype(jnp.int32)
    idx3 = idx32.reshape(NBLK, 1, KN)
    x_t = node_vectors.transpose(1, 0, 2)
    gv_t = global_vectors.transpose(1, 0, 2)

    ng_t, enh_t = _fused(x_t, idx3, gv_t, W_global, W_node)
    return enh_t.transpose(1, 0, 2), ng_t.transpose(1, 0, 2)


# speedup vs baseline: 29.6918x; 1.0064x over previous
import jax
import jax.numpy as jnp
from jax import lax
from jax.experimental import pallas as pl

N = 50000
B = 256
H = 256

KN = 2000
NBLK = N // KN
KP = 5000
NBLKP = N // KP


def _onehot_bf16(idxb):
    oh = (idxb[:, None] == lax.broadcasted_iota(jnp.int32, (1, B), 1))
    return oh.astype(jnp.bfloat16)


def _pass1_body(x_ref, idx_ref, sums_ref, cnt_ref):
    i = pl.program_id(0)

    p = _onehot_bf16(idx_ref[0, 0, :])
    ones = jnp.ones((KP, 8), jnp.bfloat16)
    c = lax.dot_general(p, ones, (((0,), (0,)), ((), ())),
                        preferred_element_type=jnp.float32)

    @pl.when(i == 0)
    def _():
        for j in range(3):
            xc = x_ref[j].astype(jnp.bfloat16)
            sums_ref[j] = lax.dot_general(p, xc, (((0,), (0,)), ((), ())),
                                          preferred_element_type=jnp.float32)
        cnt_ref[...] = c

    @pl.when(i > 0)
    def _():
        for j in range(3):
            xc = x_ref[j].astype(jnp.bfloat16)
            sums_ref[j] += lax.dot_general(p, xc, (((0,), (0,)), ((), ())),
                                           preferred_element_type=jnp.float32)
        cnt_ref[...] += c


_pass1 = pl.pallas_call(
    _pass1_body,
    grid=(NBLKP,),
    in_specs=[
        pl.BlockSpec((3, KP, H), lambda i: (0, i, 0)),
        pl.BlockSpec((1, 1, KP), lambda i: (i, 0, 0)),
    ],
    out_specs=(
        pl.BlockSpec((3, B, H), lambda i: (0, 0, 0)),
        pl.BlockSpec((B, 8), lambda i: (0, 0)),
    ),
    out_shape=(
        jax.ShapeDtypeStruct((3, B, H), jnp.float32),
        jax.ShapeDtypeStruct((B, 8), jnp.float32),
    ),
)


def _prep_body(sums_ref, cnt_ref, gv_ref, wg_ref, wn_ref, ng_ref, gw_ref):
    inv = 1.0 / jnp.maximum(cnt_ref[:, 0:1], 1.0)
    wg = wg_ref[...]
    wn = wn_ref[...]
    for j in range(3):
        tj = sums_ref[j] * inv + gv_ref[j]
        ngj = lax.dot_general(tj, wg, (((1,), (1,)), ((), ())),
                              preferred_element_type=jnp.float32)
        ng_ref[j] = ngj
        gw_ref[j] = lax.dot_general(ngj, wn, (((1,), (1,)), ((), ())),
                                    preferred_element_type=jnp.float32)


_prep = pl.pallas_call(
    _prep_body,
    out_shape=(
        jax.ShapeDtypeStruct((3, B, H), jnp.float32),
        jax.ShapeDtypeStruct((3, B, H), jnp.float32),
    ),
)


def _main_body(x_ref, idx_ref, gw_ref, wn_ref, o_ref):
    p = _onehot_bf16(idx_ref[0, 0, :])
    wn = wn_ref[...].astype(jnp.bfloat16)
    for j in range(3):
        xj = x_ref[j]
        mm = lax.dot_general(xj.astype(jnp.bfloat16), wn,
                             (((1,), (1,)), ((), ())),
                             preferred_element_type=jnp.float32)
        g = lax.dot_general(p, gw_ref[j].astype(jnp.bfloat16),
                            (((1,), (0,)), ((), ())),
                            preferred_element_type=jnp.float32)
        o_ref[j] = mm + xj + g


_main = pl.pallas_call(
    _main_body,
    grid=(NBLK,),
    in_specs=[
        pl.BlockSpec((3, KN, H), lambda i: (0, i, 0)),
        pl.BlockSpec((1, 1, KN), lambda i: (i, 0, 0)),
        pl.BlockSpec((3, B, H), lambda i: (0, 0, 0)),
        pl.BlockSpec((H, H), lambda i: (0, 0)),
    ],
    out_specs=pl.BlockSpec((3, KN, H), lambda i: (0, i, 0)),
    out_shape=jax.ShapeDtypeStruct((3, N, H), jnp.float32),
)


def kernel(node_vectors, batch_idx, global_vectors, W_node, W_global):
    idx32 = batch_idx.astype(jnp.int32)
    idx3 = idx32.reshape(NBLK, 1, KN)
    idx3p = idx32.reshape(NBLKP, 1, KP)
    x_t = node_vectors.transpose(1, 0, 2)
    gv_t = global_vectors.transpose(1, 0, 2)

    sums, cnts = _pass1(x_t, idx3p)
    ng_t, gw_t = _prep(sums, cnts, gv_t, W_global, W_node)
    enh_t = _main(x_t, idx3, gw_t, W_node)
    return enh_t.transpose(1, 0, 2), ng_t.transpose(1, 0, 2)
